# baseline (speedup 1.0000x reference)
import functools

import jax
import jax.numpy as jnp
from jax import lax
from jax.experimental import pallas as pl
from jax.experimental.pallas import tpu as pltpu
from jax.experimental.pallas import tpu_sc as plsc

_V = 1000
_B = 1024
_T = 50
_TP = 56
_NW = 32
_B_PER_W = _B // _NW


def _sc_gather(table, idxp):
  mesh = plsc.VectorSubcoreMesh(core_axis_name="c", subcore_axis_name="s")

  @functools.partial(
      pl.kernel,
      mesh=mesh,
      out_type=jax.ShapeDtypeStruct((_B, _T, _V), jnp.float32),
      compiler_params=pltpu.CompilerParams(use_tc_tiling_on_sc=False),
      scratch_types=[
          pltpu.VMEM((_B_PER_W * _TP,), jnp.int32),
          pltpu.VMEM((_T, _V), jnp.float32),
          pltpu.VMEM((_T, _V), jnp.float32),
          pltpu.SemaphoreType.DMA,
          pltpu.SemaphoreType.DMA,
          pltpu.SemaphoreType.DMA,
          pltpu.SemaphoreType.DMA,
      ],
  )
  def k(table_hbm, idx_hbm, out_hbm, idx_v, rows0, rows1,
        gsem0, gsem1, ssem0, ssem1):
    wid = lax.axis_index("s") * 2 + lax.axis_index("c")
    b0 = wid * _B_PER_W

    pltpu.sync_copy(idx_hbm.at[pl.ds(b0 * _TP, _B_PER_W * _TP)], idx_v)

    def start_gather(i, buf, sem):
      pltpu.async_copy(table_hbm.at[idx_v.at[pl.ds(i * _TP, _T)]], buf, sem)

    def wait_gather(buf, sem):
      pltpu.make_async_copy(table_hbm.at[idx_v.at[pl.ds(0, _T)]], buf,
                            sem).wait()

    def start_scatter(i, buf, sem):
      pltpu.async_copy(buf, out_hbm.at[b0 + i], sem)

    def wait_scatter(buf, sem):
      pltpu.make_async_copy(buf, out_hbm.at[0], sem).wait()

    start_gather(0, rows0, gsem0)
    start_gather(1, rows1, gsem1)

    def body(p, carry):
      i = 2 * p
      wait_gather(rows0, gsem0)
      start_scatter(i, rows0, ssem0)
      wait_gather(rows1, gsem1)
      start_scatter(i + 1, rows1, ssem1)
      wait_scatter(rows0, ssem0)
      start_gather(i + 2, rows0, gsem0)
      wait_scatter(rows1, ssem1)
      start_gather(i + 3, rows1, gsem1)
      return carry

    lax.fori_loop(0, _B_PER_W // 2 - 1, body, 0)

    wait_gather(rows0, gsem0)
    start_scatter(_B_PER_W - 2, rows0, ssem0)
    wait_gather(rows1, gsem1)
    start_scatter(_B_PER_W - 1, rows1, ssem1)
    wait_scatter(rows0, ssem0)
    wait_scatter(rows1, ssem1)

  return k(table, idxp)


def kernel(x, prob):
  idxp = jnp.pad(x.astype(jnp.int32), ((0, 0), (0, _TP - _T))).reshape(-1)
  return _sc_gather(prob, idxp)
